# baseline (speedup 1.0000x reference)
import functools

import jax
import jax.numpy as jnp
from jax import lax
from jax.experimental import pallas as pl
from jax.experimental.pallas import tpu as pltpu
from jax.experimental.pallas import tpu_sc as plsc

_B = 16384
_D = 64
_NC = 2
_NS = 16
_NW = _NC * _NS
_BPW = _B // _NW
_CHUNK = 128
_NCHUNK = _BPW // _CHUNK


def _body(feat_hbm, lab_hbm, cent_hbm, out_hbm, idx_v, rows_v, feat_v,
          part_v, gsem, fsem):
    wid = lax.axis_index("s") * _NC + lax.axis_index("c")
    base = wid * _BPW

    pltpu.sync_copy(lab_hbm.at[pl.ds(wid * _NCHUNK, _NCHUNK)], idx_v)

    fcopy = pltpu.async_copy(feat_hbm.at[pl.ds(base, _BPW)], feat_v, fsem)

    copies = []
    for j in range(_NCHUNK):
        copies.append(
            pltpu.async_copy(
                cent_hbm.at[idx_v.at[j]],
                rows_v.at[pl.ds(j * _CHUNK, _CHUNK)],
                gsem,
            ))
    fcopy.wait()
    for c in copies:
        c.wait()

    def step(i, accs):
        a0, a1, a2, a3 = accs
        d0 = feat_v[i, pl.ds(0, 16)] - rows_v[i, pl.ds(0, 16)]
        d1 = feat_v[i, pl.ds(16, 16)] - rows_v[i, pl.ds(16, 16)]
        d2 = feat_v[i, pl.ds(32, 16)] - rows_v[i, pl.ds(32, 16)]
        d3 = feat_v[i, pl.ds(48, 16)] - rows_v[i, pl.ds(48, 16)]
        return (a0 + d0 * d0, a1 + d1 * d1, a2 + d2 * d2, a3 + d3 * d3)

    zero = jnp.zeros((16,), jnp.float32)
    a0, a1, a2, a3 = lax.fori_loop(0, _BPW, step, (zero, zero, zero, zero))
    part_v[...] = (a0 + a1) + (a2 + a3)
    pltpu.sync_copy(part_v, out_hbm.at[wid])


@jax.jit
def kernel(features, labels, centers):
    labels = labels.astype(jnp.int32).reshape(_NW * _NCHUNK, _CHUNK)
    mesh = plsc.VectorSubcoreMesh(core_axis_name="c", subcore_axis_name="s")
    partials = pl.kernel(
        _body,
        out_type=jax.ShapeDtypeStruct((_NW, 16), jnp.float32),
        mesh=mesh,
        scratch_types=[
            pltpu.VMEM((_NCHUNK, _CHUNK), jnp.int32),
            pltpu.VMEM((_BPW, _D), jnp.float32),
            pltpu.VMEM((_BPW, _D), jnp.float32),
            pltpu.VMEM((16,), jnp.float32),
            pltpu.SemaphoreType.DMA,
            pltpu.SemaphoreType.DMA,
        ],
        compiler_params=pltpu.CompilerParams(use_tc_tiling_on_sc=False),
    )(features, labels, centers)
    return jnp.sum(partials) * (1.0 / (_B * _D))
